# Initial kernel scaffold; baseline (speedup 1.0000x reference)
import jax
import jax.numpy as jnp
from jax.experimental import pallas as pl


def kernel(user_table, item_table, edge_index):
    raise NotImplementedError("write your pallas kernel here")


# speedup vs baseline: 5.4185x; 5.4185x over previous
import jax
import jax.numpy as jnp
from jax import lax
from jax.experimental import pallas as pl
from jax.experimental.pallas import tpu as pltpu
from jax.experimental.pallas import tpu_sc as plsc

NUM_USER = 5000
NUM_ITEM = 5000
D = 128
E = 320000
NC = 2
NS = 16
NW = NC * NS
EPW = E // NW
CHUNK = 80
NITER = EPW // CHUNK
CW = 128
ZROWS = 200
NZCH = NUM_ITEM // ZROWS


def _phase1_body(user_hbm, src_hbm, dst_hbm, zs_hbm,
                 psum_hbm, pcnt_hbm,
                 idx_s, idx_d, rows, ones_v, acc, cnt, sem):
    cid = lax.axis_index("c")
    sid = lax.axis_index("s")
    wid = sid * NC + cid

    def fill_i(i, _):
        for g in range(CW // 16):
            ones_v[i, pl.ds(g * 16, 16)] = jnp.full((16,), 1.0, jnp.float32)
        return _

    lax.fori_loop(0, CHUNK, fill_i, None)

    def zero_k(k, _):
        ch = sid + NS * k

        @pl.when(ch < NZCH)
        def _():
            pltpu.sync_copy(zs_hbm, acc.at[pl.ds(ch * ZROWS, ZROWS), :])
            pltpu.sync_copy(zs_hbm, cnt.at[pl.ds(ch * ZROWS, ZROWS), :])
        return _

    lax.fori_loop(0, (NZCH + NS - 1) // NS, zero_k, None)
    plsc.subcore_barrier()

    def edge_j(j, _):
        base = wid * EPW + j * CHUNK
        pltpu.sync_copy(src_hbm.at[pl.ds(base, CHUNK)], idx_s)
        pltpu.sync_copy(dst_hbm.at[pl.ds(base, CHUNK)], idx_d)
        pltpu.async_copy(user_hbm.at[idx_s], rows, sem).wait()
        pltpu.sync_copy(rows, acc.at[idx_d], add=True)
        pltpu.sync_copy(ones_v, cnt.at[idx_d], add=True)
        return _

    lax.fori_loop(0, NITER, edge_j, None)
    plsc.subcore_barrier()

    def dump_k(k, _):
        ch = sid + NS * k

        @pl.when(ch < NZCH)
        def _():
            sl = pl.ds(ch * ZROWS, ZROWS)
            pltpu.sync_copy(acc.at[sl, :], psum_hbm.at[cid, sl, :])
            pltpu.sync_copy(cnt.at[sl, :], pcnt_hbm.at[cid, sl, :])
        return _

    lax.fori_loop(0, (NZCH + NS - 1) // NS, dump_k, None)


MROWS = 40
NMCH = NUM_ITEM // MROWS


def _phase2_body(psum_hbm, pcnt_hbm, out_hbm, s0, s1, c0, c1, o):
    cid = lax.axis_index("c")
    sid = lax.axis_index("s")
    wid = sid * NC + cid

    def merge_k(k, _):
        ch = wid + NW * k

        @pl.when(ch < NMCH)
        def _():
            sl = pl.ds(ch * MROWS, MROWS)
            pltpu.sync_copy(psum_hbm.at[0, sl, :], s0)
            pltpu.sync_copy(psum_hbm.at[1, sl, :], s1)
            pltpu.sync_copy(pcnt_hbm.at[0, sl, :], c0)
            pltpu.sync_copy(pcnt_hbm.at[1, sl, :], c1)

            def row_r(r, _):
                cs = pl.ds(0, 16)
                cv = jnp.maximum(c0[r, cs] + c1[r, cs], 1.0)
                rec = 1.0 / cv
                for g in range(D // 16):
                    gs = pl.ds(g * 16, 16)
                    o[r, gs] = (s0[r, gs] + s1[r, gs]) * rec
                return _

            lax.fori_loop(0, MROWS, row_r, None)
            pltpu.sync_copy(o, out_hbm.at[sl, :])
        return _

    lax.fori_loop(0, (NMCH + NW - 1) // NW, merge_k, None)


def kernel(user_table, item_table, edge_index):
    del item_table
    src = edge_index[0]
    dst = edge_index[1]
    zs = jnp.zeros((ZROWS, D), jnp.float32)

    mesh = plsc.VectorSubcoreMesh(core_axis_name="c", subcore_axis_name="s")
    phase1 = pl.kernel(
        _phase1_body,
        mesh=mesh,
        out_type=(
            jax.ShapeDtypeStruct((NC, NUM_ITEM, D), jnp.float32),
            jax.ShapeDtypeStruct((NC, NUM_ITEM, CW), jnp.float32),
        ),
        scratch_types=[
            pltpu.VMEM((CHUNK,), jnp.int32),
            pltpu.VMEM((CHUNK,), jnp.int32),
            pltpu.VMEM((CHUNK, D), jnp.float32),
            pltpu.VMEM((CHUNK, CW), jnp.float32),
            pltpu.VMEM_SHARED((NUM_ITEM, D), jnp.float32),
            pltpu.VMEM_SHARED((NUM_ITEM, CW), jnp.float32),
            pltpu.SemaphoreType.DMA,
        ],
    )
    psum, pcnt = phase1(user_table, src, dst, zs)

    phase2 = pl.kernel(
        _phase2_body,
        mesh=mesh,
        out_type=jax.ShapeDtypeStruct((NUM_ITEM, D), jnp.float32),
        scratch_types=[
            pltpu.VMEM((MROWS, D), jnp.float32),
            pltpu.VMEM((MROWS, D), jnp.float32),
            pltpu.VMEM((MROWS, CW), jnp.float32),
            pltpu.VMEM((MROWS, CW), jnp.float32),
            pltpu.VMEM((MROWS, D), jnp.float32),
        ],
    )
    return phase2(psum, pcnt)


# speedup vs baseline: 9.4599x; 1.7459x over previous
import jax
import jax.numpy as jnp
from jax import lax
from jax.experimental import pallas as pl
from jax.experimental.pallas import tpu as pltpu
from jax.experimental.pallas import tpu_sc as plsc

NUM_USER = 5000
NUM_ITEM = 5000
D = 128
E = 320000
NC = 2
NS = 16
NW = NC * NS
EPW = E // NW
CHUNK = 80
NITER = EPW // CHUNK
ZROWS = 200
NZCH = NUM_ITEM // ZROWS


def _phase1_body(user_hbm, src_hbm, dst_hbm, zs_hbm,
                 psum_hbm, pcnt_hbm,
                 is0, is1, id0, id1, rows0, rows1, ones_v,
                 acc, cnt, sem, semi):
    cid = lax.axis_index("c")
    sid = lax.axis_index("s")
    wid = sid * NC + cid

    def fill_i(i, _):
        for g in range(D // 16):
            ones_v[i, pl.ds(g * 16, 16)] = jnp.full((16,), 1.0, jnp.float32)
        return _

    lax.fori_loop(0, CHUNK, fill_i, None)

    def zero_k(k, _):
        ch = sid + NS * k

        @pl.when(ch < NZCH)
        def _():
            pltpu.sync_copy(zs_hbm, acc.at[pl.ds(ch * ZROWS, ZROWS), :])
            pltpu.sync_copy(zs_hbm, cnt.at[pl.ds(ch * ZROWS, ZROWS), :])
        return _

    lax.fori_loop(0, (NZCH + NS - 1) // NS, zero_k, None)
    plsc.subcore_barrier()

    def scat(idb, rbuf):
        pltpu.sync_copy(rbuf, acc.at[idb], add=True)
        pltpu.sync_copy(ones_v, cnt.at[idb], add=True)

    def gath(isb, rbuf):
        pltpu.async_copy(user_hbm.at[isb], rbuf, sem)

    def gwait(isb, rbuf):
        pltpu.make_async_copy(user_hbm.at[isb], rbuf, sem).wait()

    def isl(j):
        return pl.ds(wid * EPW + j * CHUNK, CHUNK)

    def idx_stage(j, isb, idb):
        pltpu.async_copy(src_hbm.at[isl(j)], isb, semi)
        pltpu.async_copy(dst_hbm.at[isl(j)], idb, semi)

    def idx_wait(j, isb, idb):
        pltpu.make_async_copy(src_hbm.at[isl(j)], isb, semi).wait()
        pltpu.make_async_copy(dst_hbm.at[isl(j)], idb, semi).wait()

    pltpu.sync_copy(src_hbm.at[isl(0)], is0)
    pltpu.sync_copy(dst_hbm.at[isl(0)], id0)
    gath(is0, rows0)
    idx_stage(1, is1, id1)

    def edge_g(g, _):
        j = 2 * g
        gwait(is0, rows0)
        idx_wait(j + 1, is1, id1)
        gath(is1, rows1)
        scat(id0, rows0)

        @pl.when(j + 2 < NITER)
        def _():
            idx_stage(j + 2, is0, id0)
        gwait(is1, rows1)

        @pl.when(j + 2 < NITER)
        def _():
            idx_wait(j + 2, is0, id0)
            gath(is0, rows0)
        scat(id1, rows1)

        @pl.when(j + 3 < NITER)
        def _():
            idx_stage(j + 3, is1, id1)
        return _

    lax.fori_loop(0, (NITER - 1) // 2, edge_g, None)
    gwait(is0, rows0)
    scat(id0, rows0)

    plsc.subcore_barrier()

    def dump_k(k, _):
        ch = sid + NS * k

        @pl.when(ch < NZCH)
        def _():
            sl = pl.ds(ch * ZROWS, ZROWS)
            pltpu.sync_copy(acc.at[sl, :], psum_hbm.at[cid, sl, :])
            pltpu.sync_copy(cnt.at[sl, :], pcnt_hbm.at[cid, sl, :])
        return _

    lax.fori_loop(0, (NZCH + NS - 1) // NS, dump_k, None)


MROWS = 40
NMCH = NUM_ITEM // MROWS


def _phase2_body(psum_hbm, pcnt_hbm, out_hbm, s0, s1, c0, c1, o):
    cid = lax.axis_index("c")
    sid = lax.axis_index("s")
    wid = sid * NC + cid

    def merge_k(k, _):
        ch = wid + NW * k

        @pl.when(ch < NMCH)
        def _():
            sl = pl.ds(ch * MROWS, MROWS)
            pltpu.sync_copy(psum_hbm.at[0, sl, :], s0)
            pltpu.sync_copy(psum_hbm.at[1, sl, :], s1)
            pltpu.sync_copy(pcnt_hbm.at[0, sl, :], c0)
            pltpu.sync_copy(pcnt_hbm.at[1, sl, :], c1)

            def row_r(r, _):
                cs = pl.ds(0, 16)
                cv = jnp.maximum(c0[r, cs] + c1[r, cs], 1.0)
                rec = 1.0 / cv
                for g in range(D // 16):
                    gs = pl.ds(g * 16, 16)
                    o[r, gs] = (s0[r, gs] + s1[r, gs]) * rec
                return _

            lax.fori_loop(0, MROWS, row_r, None)
            pltpu.sync_copy(o, out_hbm.at[sl, :])
        return _

    lax.fori_loop(0, (NMCH + NW - 1) // NW, merge_k, None)


def kernel(user_table, item_table, edge_index):
    del item_table
    src = edge_index[0]
    dst = edge_index[1]
    zs = jnp.zeros((ZROWS, D), jnp.float32)

    mesh = plsc.VectorSubcoreMesh(core_axis_name="c", subcore_axis_name="s")
    phase1 = pl.kernel(
        _phase1_body,
        mesh=mesh,
        out_type=(
            jax.ShapeDtypeStruct((NC, NUM_ITEM, D), jnp.float32),
            jax.ShapeDtypeStruct((NC, NUM_ITEM, D), jnp.float32),
        ),
        scratch_types=[
            pltpu.VMEM((CHUNK,), jnp.int32),
            pltpu.VMEM((CHUNK,), jnp.int32),
            pltpu.VMEM((CHUNK,), jnp.int32),
            pltpu.VMEM((CHUNK,), jnp.int32),
            pltpu.VMEM((CHUNK, D), jnp.float32),
            pltpu.VMEM((CHUNK, D), jnp.float32),
            pltpu.VMEM((CHUNK, D), jnp.float32),
            pltpu.VMEM_SHARED((NUM_ITEM, D), jnp.float32),
            pltpu.VMEM_SHARED((NUM_ITEM, D), jnp.float32),
            pltpu.SemaphoreType.DMA,
            pltpu.SemaphoreType.DMA,
        ],
    )
    psum, pcnt = phase1(user_table, src, dst, zs)

    phase2 = pl.kernel(
        _phase2_body,
        mesh=mesh,
        out_type=jax.ShapeDtypeStruct((NUM_ITEM, D), jnp.float32),
        scratch_types=[
            pltpu.VMEM((MROWS, D), jnp.float32),
            pltpu.VMEM((MROWS, D), jnp.float32),
            pltpu.VMEM((MROWS, D), jnp.float32),
            pltpu.VMEM((MROWS, D), jnp.float32),
            pltpu.VMEM((MROWS, D), jnp.float32),
        ],
    )
    return phase2(psum, pcnt)


# speedup vs baseline: 10.1738x; 1.0755x over previous
import jax
import jax.numpy as jnp
from jax import lax
from jax.experimental import pallas as pl
from jax.experimental.pallas import tpu as pltpu
from jax.experimental.pallas import tpu_sc as plsc

NUM_USER = 5000
NUM_ITEM = 5000
D = 128
E = 320000
NC = 2
NS = 16
NW = NC * NS
EPW = E // NW
CHUNK = 80
NITER = EPW // CHUNK
ZROWS = 200
NZCH = NUM_ITEM // ZROWS


def _phase1_body(user_hbm, src_hbm, dst_hbm, zs_hbm,
                 psum_hbm, pcnt_hbm,
                 is0, is1, id0, id1, rows0, rows1, ones_v,
                 acc, cnt, sem, semi):
    cid = lax.axis_index("c")
    sid = lax.axis_index("s")
    wid = sid * NC + cid

    def fill_i(i, _):
        for g in range(D // 16):
            ones_v[i, pl.ds(g * 16, 16)] = jnp.full((16,), 1.0, jnp.float32)
        return _

    lax.fori_loop(0, CHUNK, fill_i, None)

    def zero_k(k, _):
        ch = sid + NS * k

        @pl.when(ch < NZCH)
        def _():
            pltpu.sync_copy(zs_hbm, acc.at[pl.ds(ch * ZROWS, ZROWS), :])
            pltpu.sync_copy(zs_hbm, cnt.at[pl.ds(ch * ZROWS, ZROWS), :])
        return _

    lax.fori_loop(0, (NZCH + NS - 1) // NS, zero_k, None)
    plsc.subcore_barrier()

    def scat(idb, rbuf):
        pltpu.sync_copy(rbuf, acc.at[idb], add=True)
        pltpu.sync_copy(ones_v, cnt.at[idb], add=True)

    def gath(isb, rbuf):
        pltpu.async_copy(user_hbm.at[isb], rbuf, sem)

    def gwait(isb, rbuf):
        pltpu.make_async_copy(user_hbm.at[isb], rbuf, sem).wait()

    def isl(j):
        return pl.ds(wid * EPW + j * CHUNK, CHUNK)

    def idx_stage(j, isb, idb):
        pltpu.async_copy(src_hbm.at[isl(j)], isb, semi)
        pltpu.async_copy(dst_hbm.at[isl(j)], idb, semi)

    def idx_wait(j, isb, idb):
        pltpu.make_async_copy(src_hbm.at[isl(j)], isb, semi).wait()
        pltpu.make_async_copy(dst_hbm.at[isl(j)], idb, semi).wait()

    pltpu.sync_copy(src_hbm.at[isl(0)], is0)
    pltpu.sync_copy(dst_hbm.at[isl(0)], id0)
    gath(is0, rows0)
    idx_stage(1, is1, id1)

    def edge_g(g, _):
        j = 2 * g
        gwait(is0, rows0)
        idx_wait(j + 1, is1, id1)
        gath(is1, rows1)
        scat(id0, rows0)

        @pl.when(j + 2 < NITER)
        def _():
            idx_stage(j + 2, is0, id0)
        gwait(is1, rows1)

        @pl.when(j + 2 < NITER)
        def _():
            idx_wait(j + 2, is0, id0)
            gath(is0, rows0)
        scat(id1, rows1)

        @pl.when(j + 3 < NITER)
        def _():
            idx_stage(j + 3, is1, id1)
        return _

    lax.fori_loop(0, (NITER - 1) // 2, edge_g, None)
    gwait(is0, rows0)
    scat(id0, rows0)

    plsc.subcore_barrier()

    def dump_k(k, _):
        ch = sid + NS * k

        @pl.when(ch < NZCH)
        def _():
            sl = pl.ds(ch * ZROWS, ZROWS)
            pltpu.sync_copy(acc.at[sl, :], psum_hbm.at[cid, sl, :])
            pltpu.sync_copy(cnt.at[sl, :], pcnt_hbm.at[cid, sl, :])
        return _

    lax.fori_loop(0, (NZCH + NS - 1) // NS, dump_k, None)


MROWS = 1000


def _merge_tc_body(ps_ref, pc_ref, o_ref):
    s = ps_ref[0] + ps_ref[1]
    c = jnp.maximum(pc_ref[0] + pc_ref[1], 1.0)
    o_ref[...] = s / c


def kernel(user_table, item_table, edge_index):
    del item_table
    src = edge_index[0]
    dst = edge_index[1]
    zs = jnp.zeros((ZROWS, D), jnp.float32)

    mesh = plsc.VectorSubcoreMesh(core_axis_name="c", subcore_axis_name="s")
    phase1 = pl.kernel(
        _phase1_body,
        mesh=mesh,
        out_type=(
            jax.ShapeDtypeStruct((NC, NUM_ITEM, D), jnp.float32),
            jax.ShapeDtypeStruct((NC, NUM_ITEM, D), jnp.float32),
        ),
        scratch_types=[
            pltpu.VMEM((CHUNK,), jnp.int32),
            pltpu.VMEM((CHUNK,), jnp.int32),
            pltpu.VMEM((CHUNK,), jnp.int32),
            pltpu.VMEM((CHUNK,), jnp.int32),
            pltpu.VMEM((CHUNK, D), jnp.float32),
            pltpu.VMEM((CHUNK, D), jnp.float32),
            pltpu.VMEM((CHUNK, D), jnp.float32),
            pltpu.VMEM_SHARED((NUM_ITEM, D), jnp.float32),
            pltpu.VMEM_SHARED((NUM_ITEM, D), jnp.float32),
            pltpu.SemaphoreType.DMA,
            pltpu.SemaphoreType.DMA,
        ],
    )
    psum, pcnt = phase1(user_table, src, dst, zs)

    merge = pl.pallas_call(
        _merge_tc_body,
        grid=(NUM_ITEM // MROWS,),
        in_specs=[
            pl.BlockSpec((NC, MROWS, D), lambda i: (0, i, 0)),
            pl.BlockSpec((NC, MROWS, D), lambda i: (0, i, 0)),
        ],
        out_specs=pl.BlockSpec((MROWS, D), lambda i: (i, 0)),
        out_shape=jax.ShapeDtypeStruct((NUM_ITEM, D), jnp.float32),
    )
    return merge(psum, pcnt)
